# baseline (speedup 1.0000x reference)
import functools

import jax
import jax.numpy as jnp
from jax import lax
from jax.experimental import pallas as pl
from jax.experimental.pallas import tpu as pltpu
from jax.experimental.pallas import tpu_sc as plsc

N_NODES = 10000
N_EDGES = 320000
D = 128
N_CLASSES = 16
N_PAIRS = 4096

NC = 2
NS = 16
NW = NC * NS
CHUNK = 128
NCHUNK = 80
HALF = NCHUNK // 2
EPW = NCHUNK * CHUNK
NPAD = 10112
RPS = NPAD // NS

_MESH = plsc.VectorSubcoreMesh(
    core_axis_name="c", subcore_axis_name="s", num_cores=NC, num_subcores=NS)

f32 = jnp.float32
i32 = jnp.int32


def _fill_1d(ref, n, val):
  v = jnp.full((16,), val, f32)
  off = 0
  while off + 16 <= n:
    ref[pl.ds(off, 16)] = v
    off += 16
  if off < n:
    ref[pl.ds(n - 16, 16)] = v


_RPS_PIECES = [(o, min(CHUNK, RPS - o)) for o in range(0, RPS, CHUNK)]


def _deg_body(srcI, dstI, outO, outI, idxs_v, idxd_v, ones_v, stage, hs, hd):
  c = lax.axis_index("c")
  s = lax.axis_index("s")
  w = s * NC + c
  sub0 = s * RPS
  _fill_1d(stage, RPS, 0.0)
  _fill_1d(ones_v, CHUNK, 1.0)
  pltpu.sync_copy(stage, hs.at[pl.ds(sub0, RPS)])
  pltpu.sync_copy(stage, hd.at[pl.ds(sub0, RPS)])
  pltpu.sync_copy(srcI.at[w], idxs_v)
  pltpu.sync_copy(dstI.at[w], idxd_v)
  plsc.subcore_barrier()

  def step(j, carry):
    pltpu.sync_copy(ones_v, hs.at[idxs_v.at[j]], add=True)
    pltpu.sync_copy(ones_v, hd.at[idxd_v.at[j]], add=True)
    return carry

  lax.fori_loop(0, NCHUNK, step, 0)
  plsc.subcore_barrier()
  pltpu.sync_copy(hs.at[pl.ds(sub0, RPS)], stage)
  pltpu.sync_copy(stage, outO.at[pl.ds(c * NPAD + sub0, RPS)])
  pltpu.sync_copy(hd.at[pl.ds(sub0, RPS)], stage)
  pltpu.sync_copy(stage, outI.at[pl.ds(c * NPAD + sub0, RPS)])


_deg_kernel = pl.kernel(
    _deg_body,
    out_type=(jax.ShapeDtypeStruct((NC * NPAD,), f32),
              jax.ShapeDtypeStruct((NC * NPAD,), f32)),
    mesh=_MESH,
    scratch_types=[
        pltpu.VMEM((NCHUNK, CHUNK), i32),
        pltpu.VMEM((NCHUNK, CHUNK), i32),
        pltpu.VMEM((CHUNK,), f32),
        pltpu.VMEM((RPS,), f32),
        pltpu.VMEM_SHARED((NPAD,), f32),
        pltpu.VMEM_SHARED((NPAD,), f32),
    ],
)


def _mp_body(tab, srcI, dstI, parts, idxs_v, idxd_v, buf0, buf1, acc, sem0,
             sem1):
  c = lax.axis_index("c")
  s = lax.axis_index("s")
  w = s * NC + c
  sub0 = s * RPS

  def zrow(r, carry):
    for cc in range(D // 16):
      buf0[r, pl.ds(cc * 16, 16)] = jnp.zeros((16,), f32)
    return carry

  lax.fori_loop(0, CHUNK, zrow, 0)
  for off, sz in _RPS_PIECES:
    pltpu.sync_copy(buf0.at[pl.ds(0, sz)], acc.at[pl.ds(sub0 + off, sz)])
  plsc.subcore_barrier()

  for p in range(NCHUNK // HALF):
    pltpu.sync_copy(srcI.at[w, pl.ds(p * HALF, HALF)], idxs_v)
    pltpu.sync_copy(dstI.at[w, pl.ds(p * HALF, HALF)], idxd_v)
    pltpu.async_copy(tab.at[idxs_v.at[0]], buf0, sem0)

    def pair(jj, carry):
      j0 = 2 * jj
      pltpu.async_copy(tab.at[idxs_v.at[j0 + 1]], buf1, sem1)
      pltpu.make_async_copy(tab.at[idxs_v.at[j0]], buf0, sem0).wait()
      pltpu.sync_copy(buf0, acc.at[idxd_v.at[j0]], add=True)
      pltpu.async_copy(tab.at[idxs_v.at[j0 + 2]], buf0, sem0)
      pltpu.make_async_copy(tab.at[idxs_v.at[j0 + 1]], buf1, sem1).wait()
      pltpu.sync_copy(buf1, acc.at[idxd_v.at[j0 + 1]], add=True)
      return carry

    lax.fori_loop(0, HALF // 2 - 1, pair, 0)
    pltpu.async_copy(tab.at[idxs_v.at[HALF - 1]], buf1, sem1)
    pltpu.make_async_copy(tab.at[idxs_v.at[HALF - 2]], buf0, sem0).wait()
    pltpu.sync_copy(buf0, acc.at[idxd_v.at[HALF - 2]], add=True)
    pltpu.make_async_copy(tab.at[idxs_v.at[HALF - 1]], buf1, sem1).wait()
    pltpu.sync_copy(buf1, acc.at[idxd_v.at[HALF - 1]], add=True)
  plsc.subcore_barrier()
  for off, sz in _RPS_PIECES:
    pltpu.sync_copy(acc.at[pl.ds(sub0 + off, sz)], buf0.at[pl.ds(0, sz)])
    pltpu.sync_copy(buf0.at[pl.ds(0, sz)], parts.at[c, pl.ds(sub0 + off, sz)])


_mp_kernel = pl.kernel(
    _mp_body,
    out_type=jax.ShapeDtypeStruct((NC, NPAD, D), f32),
    mesh=_MESH,
    scratch_types=[
        pltpu.VMEM((HALF, CHUNK), i32),
        pltpu.VMEM((HALF, CHUNK), i32),
        pltpu.VMEM((CHUNK, D), f32),
        pltpu.VMEM((CHUNK, D), f32),
        pltpu.VMEM_SHARED((NPAD, D), f32),
        pltpu.SemaphoreType.DMA,
        pltpu.SemaphoreType.DMA,
    ],
)


def _mp2_body(tab, srcI, dstI, x1I, x2I, sinH, g1p, g2p, sx1, sx2, idxs_v,
              idxd_v, buf0, buf1, sbuf, acc, sem0, sem1):
  c = lax.axis_index("c")
  s = lax.axis_index("s")
  w = s * NC + c
  sub0 = s * RPS

  def zrow(r, carry):
    for cc in range(D // 16):
      buf0[r, pl.ds(cc * 16, 16)] = jnp.zeros((16,), f32)
    return carry

  lax.fori_loop(0, CHUNK, zrow, 0)
  for off, sz in _RPS_PIECES:
    pltpu.sync_copy(buf0.at[pl.ds(0, sz)], acc.at[pl.ds(sub0 + off, sz)])
  plsc.subcore_barrier()

  for p in range(NCHUNK // HALF):
    pltpu.sync_copy(srcI.at[w, pl.ds(p * HALF, HALF)], idxs_v)
    pltpu.sync_copy(dstI.at[w, pl.ds(p * HALF, HALF)], idxd_v)
    pltpu.async_copy(tab.at[idxs_v.at[0]], buf0, sem0)

    def pair(jj, carry):
      j0 = 2 * jj
      pltpu.async_copy(tab.at[idxs_v.at[j0 + 1]], buf1, sem1)
      pltpu.make_async_copy(tab.at[idxs_v.at[j0]], buf0, sem0).wait()
      pltpu.sync_copy(buf0, acc.at[idxd_v.at[j0]], add=True)
      pltpu.async_copy(tab.at[idxs_v.at[j0 + 2]], buf0, sem0)
      pltpu.make_async_copy(tab.at[idxs_v.at[j0 + 1]], buf1, sem1).wait()
      pltpu.sync_copy(buf1, acc.at[idxd_v.at[j0 + 1]], add=True)
      return carry

    lax.fori_loop(0, HALF // 2 - 1, pair, 0)
    pltpu.async_copy(tab.at[idxs_v.at[HALF - 1]], buf1, sem1)
    pltpu.make_async_copy(tab.at[idxs_v.at[HALF - 2]], buf0, sem0).wait()
    pltpu.sync_copy(buf0, acc.at[idxd_v.at[HALF - 2]], add=True)
    pltpu.make_async_copy(tab.at[idxs_v.at[HALF - 1]], buf1, sem1).wait()
    pltpu.sync_copy(buf1, acc.at[idxd_v.at[HALF - 1]], add=True)
  plsc.subcore_barrier()

  pidx = idxs_v.at[0]
  for kk in range(2):
    k0 = (s * 2 + kk) * CHUNK
    for xI, gp, sx in ((x1I, g1p, sx1), (x2I, g2p, sx2)):
      pltpu.sync_copy(xI.at[pl.ds(k0, CHUNK)], pidx)
      pltpu.async_copy(acc.at[pidx], buf0, sem0).wait()
      pltpu.sync_copy(buf0, gp.at[c, pl.ds(k0, CHUNK)])
      pltpu.async_copy(sinH.at[pidx], sbuf, sem0).wait()
      pltpu.sync_copy(sbuf, sx.at[pl.ds(c * N_PAIRS + k0, CHUNK)])


_mp2_kernel = pl.kernel(
    _mp2_body,
    out_type=(jax.ShapeDtypeStruct((NC, N_PAIRS, D), f32),
              jax.ShapeDtypeStruct((NC, N_PAIRS, D), f32),
              jax.ShapeDtypeStruct((NC * N_PAIRS,), f32),
              jax.ShapeDtypeStruct((NC * N_PAIRS,), f32)),
    mesh=_MESH,
    scratch_types=[
        pltpu.VMEM((HALF, CHUNK), i32),
        pltpu.VMEM((HALF, CHUNK), i32),
        pltpu.VMEM((CHUNK, D), f32),
        pltpu.VMEM((CHUNK, D), f32),
        pltpu.VMEM((CHUNK,), f32),
        pltpu.VMEM_SHARED((NPAD, D), f32),
        pltpu.SemaphoreType.DMA,
        pltpu.SemaphoreType.DMA,
    ],
)


def _norm_body(h_ref, dO_ref, dI_ref, hn_ref, sin_ref, sout_ref):
  s_out = lax.rsqrt(jnp.maximum(dO_ref[0] + dO_ref[1], 1.0))
  s_in = lax.rsqrt(jnp.maximum(dI_ref[0] + dI_ref[1], 1.0))
  sout_ref[...] = s_out
  sin_ref[...] = s_in
  hn_ref[0:N_NODES, :] = h_ref[...] * s_out[0:N_NODES, :]
  hn_ref[N_NODES:NPAD, :] = jnp.zeros((NPAD - N_NODES, D), f32)


_norm_kernel = pl.pallas_call(
    _norm_body,
    out_shape=(jax.ShapeDtypeStruct((NPAD, D), f32),
               jax.ShapeDtypeStruct((NPAD, 1), f32),
               jax.ShapeDtypeStruct((NPAD, 1), f32)),
)


def _layer1_body(p_ref, sin_ref, sout_ref, W_ref, b_ref, hn2_ref):
  agg = (p_ref[0] + p_ref[1]) * sin_ref[...]
  out = jnp.dot(agg, W_ref[...], preferred_element_type=f32,
                precision=jax.lax.Precision.HIGHEST) + b_ref[...]
  hn2_ref[...] = jnp.maximum(out, 0.0) * sout_ref[...]


_layer1_kernel = pl.pallas_call(
    _layer1_body,
    out_shape=jax.ShapeDtypeStruct((NPAD, D), f32),
)


def _head_body(g1p_ref, g2p_ref, sx1_ref, sx2_ref, W2_ref, b2_ref, Wl1_ref,
               bl1_ref, Wl2_ref, bl2_ref, WmT_ref, bm_ref, hc_ref, hm_ref):
  dot = functools.partial(jnp.dot, preferred_element_type=f32,
                          precision=jax.lax.Precision.HIGHEST)
  W2 = W2_ref[...]
  b2 = b2_ref[...]
  g1 = (g1p_ref[0] + g1p_ref[1]) * sx1_ref[...]
  g2 = (g2p_ref[0] + g2p_ref[1]) * sx2_ref[...]
  h1 = jnp.maximum(dot(g1, W2) + b2, 0.0)
  h2 = jnp.maximum(dot(g2, W2) + b2, 0.0)
  z = dot(h1, Wl1_ref[0:D, :])
  z += dot(h2, Wl1_ref[D:2 * D, :])
  z += dot(jnp.abs(h1 - h2), Wl1_ref[2 * D:3 * D, :])
  z = jnp.maximum(z + bl1_ref[...], 0.0)
  hc_ref[...] = dot(z, Wl2_ref[...]) + bl2_ref[...]
  hm_ref[...] = jnp.sum(z * WmT_ref[...], axis=1, keepdims=True) + bm_ref[...]


_head_kernel = pl.pallas_call(
    _head_body,
    out_shape=(jax.ShapeDtypeStruct((N_PAIRS, N_CLASSES), f32),
               jax.ShapeDtypeStruct((N_PAIRS, 1), f32)),
)


def kernel(h, edge_index, x1, x2, W1, b1, W2, b2, Wl1, bl1, Wl2, bl2, Wmse,
           bmse):
  src = edge_index[0]
  dst = edge_index[1]
  npad_e = NW * EPW - N_EDGES
  padidx = (N_NODES +
            (jnp.arange(npad_e, dtype=i32) % (NPAD - N_NODES))).astype(i32)
  srcp = jnp.concatenate([src, padidx]).reshape(NW, NCHUNK, CHUNK)
  dstp = jnp.concatenate([dst, padidx]).reshape(NW, NCHUNK, CHUNK)

  degO, degI = _deg_kernel(srcp, dstp)
  hn1, s_in, s_out = _norm_kernel(h, degO.reshape(NC, NPAD, 1),
                                  degI.reshape(NC, NPAD, 1))
  parts1 = _mp_kernel(hn1, srcp, dstp)
  hn2 = _layer1_kernel(parts1, s_in, s_out, W1, b1.reshape(1, D))
  g1p, g2p, sx1, sx2 = _mp2_kernel(hn2, srcp, dstp, x1, x2,
                                   s_in.reshape(NPAD))
  h_c, h_mse = _head_kernel(g1p, g2p, sx1[:N_PAIRS].reshape(N_PAIRS, 1),
                            sx2[:N_PAIRS].reshape(N_PAIRS, 1), W2,
                            b2.reshape(1, D), Wl1, bl1.reshape(1, D), Wl2,
                            bl2.reshape(1, N_CLASSES), Wmse.reshape(1, D),
                            bmse.reshape(1, 1))
  return (h_c, h_mse)


# speedup vs baseline: 1.1463x; 1.1463x over previous
import functools

import jax
import jax.numpy as jnp
from jax import lax
from jax.experimental import pallas as pl
from jax.experimental.pallas import tpu as pltpu
from jax.experimental.pallas import tpu_sc as plsc

N_NODES = 10000
N_EDGES = 320000
D = 128
N_CLASSES = 16
N_PAIRS = 4096

NC = 2
NS = 16
NW = NC * NS
CHUNK = 128
NCHUNK = 80
HALF = NCHUNK // 2
EPW = NCHUNK * CHUNK
NPAD = 10112
RPS = NPAD // NS

_MESH = plsc.VectorSubcoreMesh(
    core_axis_name="c", subcore_axis_name="s", num_cores=NC, num_subcores=NS)

f32 = jnp.float32
i32 = jnp.int32


def _fill_1d(ref, n, val):
  v = jnp.full((16,), val, f32)
  off = 0
  while off + 16 <= n:
    ref[pl.ds(off, 16)] = v
    off += 16
  if off < n:
    ref[pl.ds(n - 16, 16)] = v


_RPS_PIECES = [(o, min(CHUNK, RPS - o)) for o in range(0, RPS, CHUNK)]


def _deg_body(srcI, dstI, outO, outI, idxs_v, idxd_v, ones_v, stage, hs, hd):
  c = lax.axis_index("c")
  s = lax.axis_index("s")
  w = s * NC + c
  sub0 = s * RPS
  _fill_1d(stage, RPS, 0.0)
  _fill_1d(ones_v, CHUNK, 1.0)
  pltpu.sync_copy(stage, hs.at[pl.ds(sub0, RPS)])
  pltpu.sync_copy(stage, hd.at[pl.ds(sub0, RPS)])
  pltpu.sync_copy(srcI.at[w], idxs_v)
  pltpu.sync_copy(dstI.at[w], idxd_v)
  plsc.subcore_barrier()

  def step(j, carry):
    pltpu.sync_copy(ones_v, hs.at[idxs_v.at[j]], add=True)
    pltpu.sync_copy(ones_v, hd.at[idxd_v.at[j]], add=True)
    return carry

  lax.fori_loop(0, NCHUNK, step, 0)
  plsc.subcore_barrier()
  pltpu.sync_copy(hs.at[pl.ds(sub0, RPS)], stage)
  pltpu.sync_copy(stage, outO.at[pl.ds(c * NPAD + sub0, RPS)])
  pltpu.sync_copy(hd.at[pl.ds(sub0, RPS)], stage)
  pltpu.sync_copy(stage, outI.at[pl.ds(c * NPAD + sub0, RPS)])


_deg_kernel = pl.kernel(
    _deg_body,
    out_type=(jax.ShapeDtypeStruct((NC * NPAD,), f32),
              jax.ShapeDtypeStruct((NC * NPAD,), f32)),
    mesh=_MESH,
    scratch_types=[
        pltpu.VMEM((NCHUNK, CHUNK), i32),
        pltpu.VMEM((NCHUNK, CHUNK), i32),
        pltpu.VMEM((CHUNK,), f32),
        pltpu.VMEM((RPS,), f32),
        pltpu.VMEM_SHARED((NPAD,), f32),
        pltpu.VMEM_SHARED((NPAD,), f32),
    ],
)


def _mp_body(tab, srcI, dstI, parts, idxs_v, idxd_v, buf0, buf1, acc, sem0,
             sem1):
  c = lax.axis_index("c")
  s = lax.axis_index("s")
  w = s * NC + c
  sub0 = s * RPS

  def zrow(r, carry):
    for cc in range(D // 16):
      buf0[r, pl.ds(cc * 16, 16)] = jnp.zeros((16,), f32)
    return carry

  lax.fori_loop(0, CHUNK, zrow, 0)
  for off, sz in _RPS_PIECES:
    pltpu.sync_copy(buf0.at[pl.ds(0, sz)], acc.at[pl.ds(sub0 + off, sz)])
  plsc.subcore_barrier()

  for p in range(NCHUNK // HALF):
    pltpu.sync_copy(srcI.at[w, pl.ds(p * HALF, HALF)], idxs_v)
    pltpu.sync_copy(dstI.at[w, pl.ds(p * HALF, HALF)], idxd_v)
    pltpu.async_copy(tab.at[idxs_v.at[0]], buf0, sem0)

    def pair(jj, carry):
      j0 = 2 * jj
      pltpu.async_copy(tab.at[idxs_v.at[j0 + 1]], buf1, sem1)
      pltpu.make_async_copy(tab.at[idxs_v.at[j0]], buf0, sem0).wait()
      pltpu.sync_copy(buf0, acc.at[idxd_v.at[j0]], add=True)
      pltpu.async_copy(tab.at[idxs_v.at[j0 + 2]], buf0, sem0)
      pltpu.make_async_copy(tab.at[idxs_v.at[j0 + 1]], buf1, sem1).wait()
      pltpu.sync_copy(buf1, acc.at[idxd_v.at[j0 + 1]], add=True)
      return carry

    lax.fori_loop(0, HALF // 2 - 1, pair, 0)
    pltpu.async_copy(tab.at[idxs_v.at[HALF - 1]], buf1, sem1)
    pltpu.make_async_copy(tab.at[idxs_v.at[HALF - 2]], buf0, sem0).wait()
    pltpu.sync_copy(buf0, acc.at[idxd_v.at[HALF - 2]], add=True)
    pltpu.make_async_copy(tab.at[idxs_v.at[HALF - 1]], buf1, sem1).wait()
    pltpu.sync_copy(buf1, acc.at[idxd_v.at[HALF - 1]], add=True)
  plsc.subcore_barrier()
  for off, sz in _RPS_PIECES:
    pltpu.sync_copy(acc.at[pl.ds(sub0 + off, sz)], buf0.at[pl.ds(0, sz)])
    pltpu.sync_copy(buf0.at[pl.ds(0, sz)], parts.at[c, pl.ds(sub0 + off, sz)])


_mp_kernel = pl.kernel(
    _mp_body,
    out_type=jax.ShapeDtypeStruct((NC, NPAD, D), f32),
    mesh=_MESH,
    scratch_types=[
        pltpu.VMEM((HALF, CHUNK), i32),
        pltpu.VMEM((HALF, CHUNK), i32),
        pltpu.VMEM((CHUNK, D), f32),
        pltpu.VMEM((CHUNK, D), f32),
        pltpu.VMEM_SHARED((NPAD, D), f32),
        pltpu.SemaphoreType.DMA,
        pltpu.SemaphoreType.DMA,
    ],
)


def _mp2_body(tab, srcI, dstI, x1I, x2I, sinH, g1p, g2p, sx1, sx2, idxs_v,
              idxd_v, buf0, buf1, sbuf, acc, sem0, sem1):
  c = lax.axis_index("c")
  s = lax.axis_index("s")
  w = s * NC + c
  sub0 = s * RPS

  def zrow(r, carry):
    for cc in range(D // 16):
      buf0[r, pl.ds(cc * 16, 16)] = jnp.zeros((16,), f32)
    return carry

  lax.fori_loop(0, CHUNK, zrow, 0)
  for off, sz in _RPS_PIECES:
    pltpu.sync_copy(buf0.at[pl.ds(0, sz)], acc.at[pl.ds(sub0 + off, sz)])
  plsc.subcore_barrier()

  for p in range(NCHUNK // HALF):
    pltpu.sync_copy(srcI.at[w, pl.ds(p * HALF, HALF)], idxs_v)
    pltpu.sync_copy(dstI.at[w, pl.ds(p * HALF, HALF)], idxd_v)
    pltpu.async_copy(tab.at[idxs_v.at[0]], buf0, sem0)

    def pair(jj, carry):
      j0 = 2 * jj
      pltpu.async_copy(tab.at[idxs_v.at[j0 + 1]], buf1, sem1)
      pltpu.make_async_copy(tab.at[idxs_v.at[j0]], buf0, sem0).wait()
      pltpu.sync_copy(buf0, acc.at[idxd_v.at[j0]], add=True)
      pltpu.async_copy(tab.at[idxs_v.at[j0 + 2]], buf0, sem0)
      pltpu.make_async_copy(tab.at[idxs_v.at[j0 + 1]], buf1, sem1).wait()
      pltpu.sync_copy(buf1, acc.at[idxd_v.at[j0 + 1]], add=True)
      return carry

    lax.fori_loop(0, HALF // 2 - 1, pair, 0)
    pltpu.async_copy(tab.at[idxs_v.at[HALF - 1]], buf1, sem1)
    pltpu.make_async_copy(tab.at[idxs_v.at[HALF - 2]], buf0, sem0).wait()
    pltpu.sync_copy(buf0, acc.at[idxd_v.at[HALF - 2]], add=True)
    pltpu.make_async_copy(tab.at[idxs_v.at[HALF - 1]], buf1, sem1).wait()
    pltpu.sync_copy(buf1, acc.at[idxd_v.at[HALF - 1]], add=True)
  plsc.subcore_barrier()

  pidx = idxs_v.at[0]
  for kk in range(2):
    k0 = (s * 2 + kk) * CHUNK
    for xI, gp, sx in ((x1I, g1p, sx1), (x2I, g2p, sx2)):
      pltpu.sync_copy(xI.at[pl.ds(k0, CHUNK)], pidx)
      pltpu.async_copy(acc.at[pidx], buf0, sem0).wait()
      pltpu.sync_copy(buf0, gp.at[c, pl.ds(k0, CHUNK)])
      pltpu.async_copy(sinH.at[pidx], sbuf, sem0).wait()
      pltpu.sync_copy(sbuf, sx.at[pl.ds(c * N_PAIRS + k0, CHUNK)])


_mp2_kernel = pl.kernel(
    _mp2_body,
    out_type=(jax.ShapeDtypeStruct((NC, N_PAIRS, D), f32),
              jax.ShapeDtypeStruct((NC, N_PAIRS, D), f32),
              jax.ShapeDtypeStruct((NC * N_PAIRS,), f32),
              jax.ShapeDtypeStruct((NC * N_PAIRS,), f32)),
    mesh=_MESH,
    scratch_types=[
        pltpu.VMEM((HALF, CHUNK), i32),
        pltpu.VMEM((HALF, CHUNK), i32),
        pltpu.VMEM((CHUNK, D), f32),
        pltpu.VMEM((CHUNK, D), f32),
        pltpu.VMEM((CHUNK,), f32),
        pltpu.VMEM_SHARED((NPAD, D), f32),
        pltpu.SemaphoreType.DMA,
        pltpu.SemaphoreType.DMA,
    ],
)


def _norm_body(h_ref, dO_ref, dI_ref, hn_ref, sin_ref, sout_ref):
  dO = dO_ref[...]
  dI = dI_ref[...]
  s_out = lax.rsqrt(jnp.maximum(dO[0:NPAD] + dO[NPAD:2 * NPAD], 1.0))
  s_in = lax.rsqrt(jnp.maximum(dI[0:NPAD] + dI[NPAD:2 * NPAD], 1.0))
  sout_ref[...] = s_out
  sin_ref[...] = s_in
  so_col = s_out[0:N_NODES].reshape(N_NODES, 1)
  hn_ref[0:N_NODES, :] = h_ref[...] * so_col
  hn_ref[N_NODES:NPAD, :] = jnp.zeros((NPAD - N_NODES, D), f32)


_norm_kernel = pl.pallas_call(
    _norm_body,
    out_shape=(jax.ShapeDtypeStruct((NPAD, D), f32),
               jax.ShapeDtypeStruct((NPAD,), f32),
               jax.ShapeDtypeStruct((NPAD,), f32)),
)


def _layer1_body(p_ref, sin_ref, sout_ref, W_ref, b_ref, hn2_ref):
  s_in = sin_ref[...].reshape(NPAD, 1)
  s_out = sout_ref[...].reshape(NPAD, 1)
  agg = (p_ref[0] + p_ref[1]) * s_in
  out = jnp.dot(agg, W_ref[...], preferred_element_type=f32,
                precision=jax.lax.Precision.HIGHEST) + b_ref[...]
  hn2_ref[...] = jnp.maximum(out, 0.0) * s_out


_layer1_kernel = pl.pallas_call(
    _layer1_body,
    out_shape=jax.ShapeDtypeStruct((NPAD, D), f32),
)


def _head_body(g1p_ref, g2p_ref, sx1_ref, sx2_ref, W2_ref, b2_ref, Wl1_ref,
               bl1_ref, Wl2_ref, bl2_ref, WmT_ref, bm_ref, hc_ref, hm_ref):
  dot = functools.partial(jnp.dot, preferred_element_type=f32,
                          precision=jax.lax.Precision.HIGHEST)
  W2 = W2_ref[...]
  b2 = b2_ref[...]
  sx1 = sx1_ref[...][0:N_PAIRS].reshape(N_PAIRS, 1)
  sx2 = sx2_ref[...][0:N_PAIRS].reshape(N_PAIRS, 1)
  g1 = (g1p_ref[0] + g1p_ref[1]) * sx1
  g2 = (g2p_ref[0] + g2p_ref[1]) * sx2
  h1 = jnp.maximum(dot(g1, W2) + b2, 0.0)
  h2 = jnp.maximum(dot(g2, W2) + b2, 0.0)
  z = dot(h1, Wl1_ref[0:D, :])
  z += dot(h2, Wl1_ref[D:2 * D, :])
  z += dot(jnp.abs(h1 - h2), Wl1_ref[2 * D:3 * D, :])
  z = jnp.maximum(z + bl1_ref[...], 0.0)
  hc_ref[...] = dot(z, Wl2_ref[...]) + bl2_ref[...]
  hm_ref[...] = jnp.sum(z * WmT_ref[...], axis=1, keepdims=True) + bm_ref[...]


_head_kernel = pl.pallas_call(
    _head_body,
    out_shape=(jax.ShapeDtypeStruct((N_PAIRS, N_CLASSES), f32),
               jax.ShapeDtypeStruct((N_PAIRS, 1), f32)),
)


def kernel(h, edge_index, x1, x2, W1, b1, W2, b2, Wl1, bl1, Wl2, bl2, Wmse,
           bmse):
  src = edge_index[0]
  dst = edge_index[1]
  npad_e = NW * EPW - N_EDGES
  padidx = (N_NODES +
            (jnp.arange(npad_e, dtype=i32) % (NPAD - N_NODES))).astype(i32)
  srcp = jnp.concatenate([src, padidx]).reshape(NW, NCHUNK, CHUNK)
  dstp = jnp.concatenate([dst, padidx]).reshape(NW, NCHUNK, CHUNK)

  degO, degI = _deg_kernel(srcp, dstp)
  hn1, s_in, s_out = _norm_kernel(h, degO, degI)
  parts1 = _mp_kernel(hn1, srcp, dstp)
  hn2 = _layer1_kernel(parts1, s_in, s_out, W1, b1.reshape(1, D))
  g1p, g2p, sx1, sx2 = _mp2_kernel(hn2, srcp, dstp, x1, x2, s_in)
  h_c, h_mse = _head_kernel(g1p, g2p, sx1, sx2, W2,
                            b2.reshape(1, D), Wl1, bl1.reshape(1, D), Wl2,
                            bl2.reshape(1, N_CLASSES), Wmse.reshape(1, D),
                            bmse.reshape(1, 1))
  return (h_c, h_mse)


# speedup vs baseline: 1.2139x; 1.0590x over previous
import functools

import jax
import jax.numpy as jnp
from jax import lax
from jax.experimental import pallas as pl
from jax.experimental.pallas import tpu as pltpu
from jax.experimental.pallas import tpu_sc as plsc

N_NODES = 10000
N_EDGES = 320000
D = 128
N_CLASSES = 16
N_PAIRS = 4096

NC = 2
NS = 16
NW = NC * NS
CHUNK = 128
NCHUNK = 80
HALF = NCHUNK // 2
EPW = NCHUNK * CHUNK
NPAD = 10112
RPS = NPAD // NS

_MESH = plsc.VectorSubcoreMesh(
    core_axis_name="c", subcore_axis_name="s", num_cores=NC, num_subcores=NS)

f32 = jnp.float32
i32 = jnp.int32


def _fill_1d(ref, n, val):
  v = jnp.full((16,), val, f32)
  off = 0
  while off + 16 <= n:
    ref[pl.ds(off, 16)] = v
    off += 16
  if off < n:
    ref[pl.ds(n - 16, 16)] = v


_RPS_PIECES = [(o, min(CHUNK, RPS - o)) for o in range(0, RPS, CHUNK)]

HC = CHUNK // 2


def _gather2(tab, idxs_v, j, buf, sem):
  pltpu.async_copy(tab.at[idxs_v.at[j, pl.ds(0, HC)]], buf.at[pl.ds(0, HC)],
                   sem)
  pltpu.async_copy(tab.at[idxs_v.at[j, pl.ds(HC, HC)]], buf.at[pl.ds(HC, HC)],
                   sem)


def _gwait2(tab, idxs_v, j, buf, sem):
  pltpu.make_async_copy(tab.at[idxs_v.at[j, pl.ds(0, HC)]],
                        buf.at[pl.ds(0, HC)], sem).wait()
  pltpu.make_async_copy(tab.at[idxs_v.at[j, pl.ds(HC, HC)]],
                        buf.at[pl.ds(HC, HC)], sem).wait()


def _deg_body(srcI, dstI, outO, outI, idxs_v, idxd_v, ones_v, stage, hs, hd):
  c = lax.axis_index("c")
  s = lax.axis_index("s")
  w = s * NC + c
  sub0 = s * RPS
  _fill_1d(stage, RPS, 0.0)
  _fill_1d(ones_v, CHUNK, 1.0)
  pltpu.sync_copy(stage, hs.at[pl.ds(sub0, RPS)])
  pltpu.sync_copy(stage, hd.at[pl.ds(sub0, RPS)])
  pltpu.sync_copy(srcI.at[pl.ds(w * NCHUNK, NCHUNK)], idxs_v)
  pltpu.sync_copy(dstI.at[pl.ds(w * NCHUNK, NCHUNK)], idxd_v)
  plsc.subcore_barrier()

  def step(j, carry):
    pltpu.sync_copy(ones_v, hs.at[idxs_v.at[j]], add=True)
    pltpu.sync_copy(ones_v, hd.at[idxd_v.at[j]], add=True)
    return carry

  lax.fori_loop(0, NCHUNK, step, 0)
  plsc.subcore_barrier()
  pltpu.sync_copy(hs.at[pl.ds(sub0, RPS)], stage)
  pltpu.sync_copy(stage, outO.at[pl.ds(c * NPAD + sub0, RPS)])
  pltpu.sync_copy(hd.at[pl.ds(sub0, RPS)], stage)
  pltpu.sync_copy(stage, outI.at[pl.ds(c * NPAD + sub0, RPS)])


_deg_kernel = pl.kernel(
    _deg_body,
    out_type=(jax.ShapeDtypeStruct((NC * NPAD,), f32),
              jax.ShapeDtypeStruct((NC * NPAD,), f32)),
    mesh=_MESH,
    scratch_types=[
        pltpu.VMEM((NCHUNK, CHUNK), i32),
        pltpu.VMEM((NCHUNK, CHUNK), i32),
        pltpu.VMEM((CHUNK,), f32),
        pltpu.VMEM((RPS,), f32),
        pltpu.VMEM_SHARED((NPAD,), f32),
        pltpu.VMEM_SHARED((NPAD,), f32),
    ],
)


def _mp_body(tab, srcI, dstI, parts, idxs_v, idxd_v, buf0, buf1, acc, sem0,
             sem1):
  c = lax.axis_index("c")
  s = lax.axis_index("s")
  w = s * NC + c
  sub0 = s * RPS

  def zrow(r, carry):
    for cc in range(D // 16):
      buf0[r, pl.ds(cc * 16, 16)] = jnp.zeros((16,), f32)
    return carry

  lax.fori_loop(0, CHUNK, zrow, 0)
  for off, sz in _RPS_PIECES:
    pltpu.sync_copy(buf0.at[pl.ds(0, sz)], acc.at[pl.ds(sub0 + off, sz)])
  plsc.subcore_barrier()

  for p in range(NCHUNK // HALF):
    pltpu.sync_copy(srcI.at[pl.ds(w * NCHUNK + p * HALF, HALF)], idxs_v)
    pltpu.sync_copy(dstI.at[pl.ds(w * NCHUNK + p * HALF, HALF)], idxd_v)
    _gather2(tab, idxs_v, 0, buf0, sem0)

    def pair(jj, carry):
      j0 = 2 * jj
      _gather2(tab, idxs_v, j0 + 1, buf1, sem1)
      _gwait2(tab, idxs_v, j0, buf0, sem0)
      pltpu.sync_copy(buf0, acc.at[idxd_v.at[j0]], add=True)
      _gather2(tab, idxs_v, j0 + 2, buf0, sem0)
      _gwait2(tab, idxs_v, j0 + 1, buf1, sem1)
      pltpu.sync_copy(buf1, acc.at[idxd_v.at[j0 + 1]], add=True)
      return carry

    lax.fori_loop(0, HALF // 2 - 1, pair, 0)
    _gather2(tab, idxs_v, HALF - 1, buf1, sem1)
    _gwait2(tab, idxs_v, HALF - 2, buf0, sem0)
    pltpu.sync_copy(buf0, acc.at[idxd_v.at[HALF - 2]], add=True)
    _gwait2(tab, idxs_v, HALF - 1, buf1, sem1)
    pltpu.sync_copy(buf1, acc.at[idxd_v.at[HALF - 1]], add=True)
  plsc.subcore_barrier()
  for off, sz in _RPS_PIECES:
    pltpu.sync_copy(acc.at[pl.ds(sub0 + off, sz)], buf0.at[pl.ds(0, sz)])
    pltpu.sync_copy(buf0.at[pl.ds(0, sz)], parts.at[c, pl.ds(sub0 + off, sz)])


_mp_kernel = pl.kernel(
    _mp_body,
    out_type=jax.ShapeDtypeStruct((NC, NPAD, D), f32),
    mesh=_MESH,
    scratch_types=[
        pltpu.VMEM((HALF, CHUNK), i32),
        pltpu.VMEM((HALF, CHUNK), i32),
        pltpu.VMEM((CHUNK, D), f32),
        pltpu.VMEM((CHUNK, D), f32),
        pltpu.VMEM_SHARED((NPAD, D), f32),
        pltpu.SemaphoreType.DMA,
        pltpu.SemaphoreType.DMA,
    ],
)


def _mp2_body(tab, srcI, dstI, x1I, x2I, sinH, g1p, g2p, sx1, sx2, idxs_v,
              idxd_v, buf0, buf1, sbuf, acc, sem0, sem1):
  c = lax.axis_index("c")
  s = lax.axis_index("s")
  w = s * NC + c
  sub0 = s * RPS

  def zrow(r, carry):
    for cc in range(D // 16):
      buf0[r, pl.ds(cc * 16, 16)] = jnp.zeros((16,), f32)
    return carry

  lax.fori_loop(0, CHUNK, zrow, 0)
  for off, sz in _RPS_PIECES:
    pltpu.sync_copy(buf0.at[pl.ds(0, sz)], acc.at[pl.ds(sub0 + off, sz)])
  plsc.subcore_barrier()

  for p in range(NCHUNK // HALF):
    pltpu.sync_copy(srcI.at[pl.ds(w * NCHUNK + p * HALF, HALF)], idxs_v)
    pltpu.sync_copy(dstI.at[pl.ds(w * NCHUNK + p * HALF, HALF)], idxd_v)
    _gather2(tab, idxs_v, 0, buf0, sem0)

    def pair(jj, carry):
      j0 = 2 * jj
      _gather2(tab, idxs_v, j0 + 1, buf1, sem1)
      _gwait2(tab, idxs_v, j0, buf0, sem0)
      pltpu.sync_copy(buf0, acc.at[idxd_v.at[j0]], add=True)
      _gather2(tab, idxs_v, j0 + 2, buf0, sem0)
      _gwait2(tab, idxs_v, j0 + 1, buf1, sem1)
      pltpu.sync_copy(buf1, acc.at[idxd_v.at[j0 + 1]], add=True)
      return carry

    lax.fori_loop(0, HALF // 2 - 1, pair, 0)
    _gather2(tab, idxs_v, HALF - 1, buf1, sem1)
    _gwait2(tab, idxs_v, HALF - 2, buf0, sem0)
    pltpu.sync_copy(buf0, acc.at[idxd_v.at[HALF - 2]], add=True)
    _gwait2(tab, idxs_v, HALF - 1, buf1, sem1)
    pltpu.sync_copy(buf1, acc.at[idxd_v.at[HALF - 1]], add=True)
  plsc.subcore_barrier()

  pidx = idxs_v.at[0]
  for kk in range(2):
    k0 = (s * 2 + kk) * CHUNK
    for xI, gp, sx in ((x1I, g1p, sx1), (x2I, g2p, sx2)):
      pltpu.sync_copy(xI.at[pl.ds(k0, CHUNK)], pidx)
      pltpu.async_copy(acc.at[pidx], buf0, sem0).wait()
      pltpu.sync_copy(buf0, gp.at[c, pl.ds(k0, CHUNK)])
      pltpu.async_copy(sinH.at[pidx], sbuf, sem0).wait()
      pltpu.sync_copy(sbuf, sx.at[pl.ds(c * N_PAIRS + k0, CHUNK)])


_mp2_kernel = pl.kernel(
    _mp2_body,
    out_type=(jax.ShapeDtypeStruct((NC, N_PAIRS, D), f32),
              jax.ShapeDtypeStruct((NC, N_PAIRS, D), f32),
              jax.ShapeDtypeStruct((NC * N_PAIRS,), f32),
              jax.ShapeDtypeStruct((NC * N_PAIRS,), f32)),
    mesh=_MESH,
    scratch_types=[
        pltpu.VMEM((HALF, CHUNK), i32),
        pltpu.VMEM((HALF, CHUNK), i32),
        pltpu.VMEM((CHUNK, D), f32),
        pltpu.VMEM((CHUNK, D), f32),
        pltpu.VMEM((CHUNK,), f32),
        pltpu.VMEM_SHARED((NPAD, D), f32),
        pltpu.SemaphoreType.DMA,
        pltpu.SemaphoreType.DMA,
    ],
)


def _norm_body(h_ref, dO_ref, dI_ref, hn_ref, sin_ref, sout_ref):
  dO = dO_ref[...]
  dI = dI_ref[...]
  s_out = lax.rsqrt(jnp.maximum(dO[0:NPAD] + dO[NPAD:2 * NPAD], 1.0))
  s_in = lax.rsqrt(jnp.maximum(dI[0:NPAD] + dI[NPAD:2 * NPAD], 1.0))
  sout_ref[...] = s_out
  sin_ref[...] = s_in
  so_col = s_out[0:N_NODES].reshape(N_NODES, 1)
  hn_ref[0:N_NODES, :] = h_ref[...] * so_col
  hn_ref[N_NODES:NPAD, :] = jnp.zeros((NPAD - N_NODES, D), f32)


_norm_kernel = pl.pallas_call(
    _norm_body,
    out_shape=(jax.ShapeDtypeStruct((NPAD, D), f32),
               jax.ShapeDtypeStruct((NPAD,), f32),
               jax.ShapeDtypeStruct((NPAD,), f32)),
)


def _layer1_body(p_ref, sin_ref, sout_ref, W_ref, b_ref, hn2_ref):
  s_in = sin_ref[...].reshape(NPAD, 1)
  s_out = sout_ref[...].reshape(NPAD, 1)
  agg = (p_ref[0] + p_ref[1]) * s_in
  out = jnp.dot(agg, W_ref[...], preferred_element_type=f32) + b_ref[...]
  hn2_ref[...] = jnp.maximum(out, 0.0) * s_out


_layer1_kernel = pl.pallas_call(
    _layer1_body,
    out_shape=jax.ShapeDtypeStruct((NPAD, D), f32),
)


def _head_body(g1p_ref, g2p_ref, sx1_ref, sx2_ref, W2_ref, b2_ref, Wl1_ref,
               bl1_ref, Wl2_ref, bl2_ref, WmT_ref, bm_ref, hc_ref, hm_ref):
  dot = functools.partial(jnp.dot, preferred_element_type=f32)
  W2 = W2_ref[...]
  b2 = b2_ref[...]
  sx1 = sx1_ref[...][0:N_PAIRS].reshape(N_PAIRS, 1)
  sx2 = sx2_ref[...][0:N_PAIRS].reshape(N_PAIRS, 1)
  g1 = (g1p_ref[0] + g1p_ref[1]) * sx1
  g2 = (g2p_ref[0] + g2p_ref[1]) * sx2
  h1 = jnp.maximum(dot(g1, W2) + b2, 0.0)
  h2 = jnp.maximum(dot(g2, W2) + b2, 0.0)
  z = dot(h1, Wl1_ref[0:D, :])
  z += dot(h2, Wl1_ref[D:2 * D, :])
  z += dot(jnp.abs(h1 - h2), Wl1_ref[2 * D:3 * D, :])
  z = jnp.maximum(z + bl1_ref[...], 0.0)
  hc_ref[...] = dot(z, Wl2_ref[...]) + bl2_ref[...]
  hm_ref[...] = jnp.sum(z * WmT_ref[...], axis=1, keepdims=True) + bm_ref[...]


_head_kernel = pl.pallas_call(
    _head_body,
    out_shape=(jax.ShapeDtypeStruct((N_PAIRS, N_CLASSES), f32),
               jax.ShapeDtypeStruct((N_PAIRS, 1), f32)),
)


def kernel(h, edge_index, x1, x2, W1, b1, W2, b2, Wl1, bl1, Wl2, bl2, Wmse,
           bmse):
  src = edge_index[0]
  dst = edge_index[1]
  npad_e = NW * EPW - N_EDGES
  padidx = (N_NODES +
            (jnp.arange(npad_e, dtype=i32) % (NPAD - N_NODES))).astype(i32)
  srcp = jnp.concatenate([src, padidx]).reshape(NW * NCHUNK, CHUNK)
  dstp = jnp.concatenate([dst, padidx]).reshape(NW * NCHUNK, CHUNK)

  degO, degI = _deg_kernel(srcp, dstp)
  hn1, s_in, s_out = _norm_kernel(h, degO, degI)
  parts1 = _mp_kernel(hn1, srcp, dstp)
  hn2 = _layer1_kernel(parts1, s_in, s_out, W1, b1.reshape(1, D))
  g1p, g2p, sx1, sx2 = _mp2_kernel(hn2, srcp, dstp, x1, x2, s_in)
  h_c, h_mse = _head_kernel(g1p, g2p, sx1, sx2, W2,
                            b2.reshape(1, D), Wl1, bl1.reshape(1, D), Wl2,
                            bl2.reshape(1, N_CLASSES), Wmse.reshape(1, D),
                            bmse.reshape(1, 1))
  return (h_c, h_mse)


# speedup vs baseline: 1.2154x; 1.0012x over previous
import functools

import jax
import jax.numpy as jnp
from jax import lax
from jax.experimental import pallas as pl
from jax.experimental.pallas import tpu as pltpu
from jax.experimental.pallas import tpu_sc as plsc

N_NODES = 10000
N_EDGES = 320000
D = 128
N_CLASSES = 16
N_PAIRS = 4096

NC = 2
NS = 16
NW = NC * NS
CHUNK = 128
NCHUNK = 80
HALF = NCHUNK // 2
EPW = NCHUNK * CHUNK
NPAD = 10112
RPS = NPAD // NS

_MESH = plsc.VectorSubcoreMesh(
    core_axis_name="c", subcore_axis_name="s", num_cores=NC, num_subcores=NS)

f32 = jnp.float32
i32 = jnp.int32


def _fill_1d(ref, n, val):
  v = jnp.full((16,), val, f32)
  off = 0
  while off + 16 <= n:
    ref[pl.ds(off, 16)] = v
    off += 16
  if off < n:
    ref[pl.ds(n - 16, 16)] = v


_RPS_PIECES = [(o, min(CHUNK, RPS - o)) for o in range(0, RPS, CHUNK)]

HC = CHUNK // 2


def _gather2(tab, idxs_v, j, buf, sem):
  pltpu.async_copy(tab.at[idxs_v.at[j, pl.ds(0, HC)]], buf.at[pl.ds(0, HC)],
                   sem)
  pltpu.async_copy(tab.at[idxs_v.at[j, pl.ds(HC, HC)]], buf.at[pl.ds(HC, HC)],
                   sem)


def _gwait2(tab, idxs_v, j, buf, sem):
  pltpu.make_async_copy(tab.at[idxs_v.at[j, pl.ds(0, HC)]],
                        buf.at[pl.ds(0, HC)], sem).wait()
  pltpu.make_async_copy(tab.at[idxs_v.at[j, pl.ds(HC, HC)]],
                        buf.at[pl.ds(HC, HC)], sem).wait()


def _deg_body(srcI, dstI, outO, outI, idxs_v, idxd_v, ones_v, stage, hs, hd):
  c = lax.axis_index("c")
  s = lax.axis_index("s")
  w = s * NC + c
  sub0 = s * RPS
  _fill_1d(stage, RPS, 0.0)
  _fill_1d(ones_v, CHUNK, 1.0)
  pltpu.sync_copy(stage, hs.at[pl.ds(sub0, RPS)])
  pltpu.sync_copy(stage, hd.at[pl.ds(sub0, RPS)])
  pltpu.sync_copy(srcI.at[pl.ds(w * NCHUNK, NCHUNK)], idxs_v)
  pltpu.sync_copy(dstI.at[pl.ds(w * NCHUNK, NCHUNK)], idxd_v)
  plsc.subcore_barrier()

  def step(j, carry):
    pltpu.sync_copy(ones_v, hs.at[idxs_v.at[j]], add=True)
    pltpu.sync_copy(ones_v, hd.at[idxd_v.at[j]], add=True)
    return carry

  lax.fori_loop(0, NCHUNK, step, 0)
  plsc.subcore_barrier()
  pltpu.sync_copy(hs.at[pl.ds(sub0, RPS)], stage)
  pltpu.sync_copy(stage, outO.at[pl.ds(c * NPAD + sub0, RPS)])
  pltpu.sync_copy(hd.at[pl.ds(sub0, RPS)], stage)
  pltpu.sync_copy(stage, outI.at[pl.ds(c * NPAD + sub0, RPS)])


_deg_kernel = pl.kernel(
    _deg_body,
    out_type=(jax.ShapeDtypeStruct((NC * NPAD,), f32),
              jax.ShapeDtypeStruct((NC * NPAD,), f32)),
    mesh=_MESH,
    scratch_types=[
        pltpu.VMEM((NCHUNK, CHUNK), i32),
        pltpu.VMEM((NCHUNK, CHUNK), i32),
        pltpu.VMEM((CHUNK,), f32),
        pltpu.VMEM((RPS,), f32),
        pltpu.VMEM_SHARED((NPAD,), f32),
        pltpu.VMEM_SHARED((NPAD,), f32),
    ],
)


def _mp_body(tab, srcI, dstI, parts, idxs_v, idxd_v, buf0, buf1, acc, sem0,
             sem1):
  c = lax.axis_index("c")
  s = lax.axis_index("s")
  w = s * NC + c
  sub0 = s * RPS

  def zrow(r, carry):
    for cc in range(D // 16):
      buf0[r, pl.ds(cc * 16, 16)] = jnp.zeros((16,), f32)
    return carry

  lax.fori_loop(0, CHUNK, zrow, 0)
  for off, sz in _RPS_PIECES:
    pltpu.sync_copy(buf0.at[pl.ds(0, sz)], acc.at[pl.ds(sub0 + off, sz)])
  plsc.subcore_barrier()

  for p in range(NCHUNK // HALF):
    pltpu.sync_copy(srcI.at[pl.ds(w * NCHUNK + p * HALF, HALF)], idxs_v)
    pltpu.sync_copy(dstI.at[pl.ds(w * NCHUNK + p * HALF, HALF)], idxd_v)
    _gather2(tab, idxs_v, 0, buf0, sem0)

    def pair(jj, carry):
      j0 = 2 * jj
      _gather2(tab, idxs_v, j0 + 1, buf1, sem1)
      _gwait2(tab, idxs_v, j0, buf0, sem0)
      pltpu.sync_copy(buf0, acc.at[idxd_v.at[j0]], add=True)
      _gather2(tab, idxs_v, j0 + 2, buf0, sem0)
      _gwait2(tab, idxs_v, j0 + 1, buf1, sem1)
      pltpu.sync_copy(buf1, acc.at[idxd_v.at[j0 + 1]], add=True)
      return carry

    lax.fori_loop(0, HALF // 2 - 1, pair, 0)
    _gather2(tab, idxs_v, HALF - 1, buf1, sem1)
    _gwait2(tab, idxs_v, HALF - 2, buf0, sem0)
    pltpu.sync_copy(buf0, acc.at[idxd_v.at[HALF - 2]], add=True)
    _gwait2(tab, idxs_v, HALF - 1, buf1, sem1)
    pltpu.sync_copy(buf1, acc.at[idxd_v.at[HALF - 1]], add=True)
  plsc.subcore_barrier()
  pltpu.sync_copy(acc.at[pl.ds(sub0, RPS)], parts.at[c, pl.ds(sub0, RPS)])


_mp_kernel = pl.kernel(
    _mp_body,
    out_type=jax.ShapeDtypeStruct((NC, NPAD, D), f32),
    mesh=_MESH,
    scratch_types=[
        pltpu.VMEM((HALF, CHUNK), i32),
        pltpu.VMEM((HALF, CHUNK), i32),
        pltpu.VMEM((CHUNK, D), f32),
        pltpu.VMEM((CHUNK, D), f32),
        pltpu.VMEM_SHARED((NPAD, D), f32),
        pltpu.SemaphoreType.DMA,
        pltpu.SemaphoreType.DMA,
    ],
)


def _mp2_body(tab, srcI, dstI, x1I, x2I, sinH, g1p, g2p, sx1, sx2, idxs_v,
              idxd_v, buf0, buf1, sbuf, acc, sem0, sem1):
  c = lax.axis_index("c")
  s = lax.axis_index("s")
  w = s * NC + c
  sub0 = s * RPS

  def zrow(r, carry):
    for cc in range(D // 16):
      buf0[r, pl.ds(cc * 16, 16)] = jnp.zeros((16,), f32)
    return carry

  lax.fori_loop(0, CHUNK, zrow, 0)
  for off, sz in _RPS_PIECES:
    pltpu.sync_copy(buf0.at[pl.ds(0, sz)], acc.at[pl.ds(sub0 + off, sz)])
  plsc.subcore_barrier()

  for p in range(NCHUNK // HALF):
    pltpu.sync_copy(srcI.at[pl.ds(w * NCHUNK + p * HALF, HALF)], idxs_v)
    pltpu.sync_copy(dstI.at[pl.ds(w * NCHUNK + p * HALF, HALF)], idxd_v)
    _gather2(tab, idxs_v, 0, buf0, sem0)

    def pair(jj, carry):
      j0 = 2 * jj
      _gather2(tab, idxs_v, j0 + 1, buf1, sem1)
      _gwait2(tab, idxs_v, j0, buf0, sem0)
      pltpu.sync_copy(buf0, acc.at[idxd_v.at[j0]], add=True)
      _gather2(tab, idxs_v, j0 + 2, buf0, sem0)
      _gwait2(tab, idxs_v, j0 + 1, buf1, sem1)
      pltpu.sync_copy(buf1, acc.at[idxd_v.at[j0 + 1]], add=True)
      return carry

    lax.fori_loop(0, HALF // 2 - 1, pair, 0)
    _gather2(tab, idxs_v, HALF - 1, buf1, sem1)
    _gwait2(tab, idxs_v, HALF - 2, buf0, sem0)
    pltpu.sync_copy(buf0, acc.at[idxd_v.at[HALF - 2]], add=True)
    _gwait2(tab, idxs_v, HALF - 1, buf1, sem1)
    pltpu.sync_copy(buf1, acc.at[idxd_v.at[HALF - 1]], add=True)
  plsc.subcore_barrier()

  pidx = idxs_v.at[0]
  for kk in range(2):
    k0 = (s * 2 + kk) * CHUNK
    for xI, gp, sx in ((x1I, g1p, sx1), (x2I, g2p, sx2)):
      pltpu.sync_copy(xI.at[pl.ds(k0, CHUNK)], pidx)
      pltpu.async_copy(acc.at[pidx], buf0, sem0).wait()
      pltpu.sync_copy(buf0, gp.at[c, pl.ds(k0, CHUNK)])
      pltpu.async_copy(sinH.at[pidx], sbuf, sem0).wait()
      pltpu.sync_copy(sbuf, sx.at[pl.ds(c * N_PAIRS + k0, CHUNK)])


_mp2_kernel = pl.kernel(
    _mp2_body,
    out_type=(jax.ShapeDtypeStruct((NC, N_PAIRS, D), f32),
              jax.ShapeDtypeStruct((NC, N_PAIRS, D), f32),
              jax.ShapeDtypeStruct((NC * N_PAIRS,), f32),
              jax.ShapeDtypeStruct((NC * N_PAIRS,), f32)),
    mesh=_MESH,
    scratch_types=[
        pltpu.VMEM((HALF, CHUNK), i32),
        pltpu.VMEM((HALF, CHUNK), i32),
        pltpu.VMEM((CHUNK, D), f32),
        pltpu.VMEM((CHUNK, D), f32),
        pltpu.VMEM((CHUNK,), f32),
        pltpu.VMEM_SHARED((NPAD, D), f32),
        pltpu.SemaphoreType.DMA,
        pltpu.SemaphoreType.DMA,
    ],
)


def _norm_body(h_ref, dO_ref, dI_ref, hn_ref, sin_ref, sout_ref):
  dO = dO_ref[...]
  dI = dI_ref[...]
  s_out = lax.rsqrt(jnp.maximum(dO[0:NPAD] + dO[NPAD:2 * NPAD], 1.0))
  s_in = lax.rsqrt(jnp.maximum(dI[0:NPAD] + dI[NPAD:2 * NPAD], 1.0))
  sout_ref[...] = s_out
  sin_ref[...] = s_in
  so_col = s_out[0:N_NODES].reshape(N_NODES, 1)
  hn_ref[0:N_NODES, :] = h_ref[...] * so_col
  hn_ref[N_NODES:NPAD, :] = jnp.zeros((NPAD - N_NODES, D), f32)


_norm_kernel = pl.pallas_call(
    _norm_body,
    out_shape=(jax.ShapeDtypeStruct((NPAD, D), f32),
               jax.ShapeDtypeStruct((NPAD,), f32),
               jax.ShapeDtypeStruct((NPAD,), f32)),
)


def _layer1_body(p_ref, sin_ref, sout_ref, W_ref, b_ref, hn2_ref):
  s_in = sin_ref[...].reshape(NPAD, 1)
  s_out = sout_ref[...].reshape(NPAD, 1)
  agg = (p_ref[0] + p_ref[1]) * s_in
  out = jnp.dot(agg, W_ref[...], preferred_element_type=f32) + b_ref[...]
  hn2_ref[...] = jnp.maximum(out, 0.0) * s_out


_layer1_kernel = pl.pallas_call(
    _layer1_body,
    out_shape=jax.ShapeDtypeStruct((NPAD, D), f32),
)


def _head_body(g1p_ref, g2p_ref, sx1_ref, sx2_ref, W2_ref, b2_ref, Wl1_ref,
               bl1_ref, Wl2_ref, bl2_ref, WmT_ref, bm_ref, hc_ref, hm_ref):
  dot = functools.partial(jnp.dot, preferred_element_type=f32)
  W2 = W2_ref[...]
  b2 = b2_ref[...]
  sx1 = sx1_ref[...][0:N_PAIRS].reshape(N_PAIRS, 1)
  sx2 = sx2_ref[...][0:N_PAIRS].reshape(N_PAIRS, 1)
  g1 = (g1p_ref[0] + g1p_ref[1]) * sx1
  g2 = (g2p_ref[0] + g2p_ref[1]) * sx2
  h1 = jnp.maximum(dot(g1, W2) + b2, 0.0)
  h2 = jnp.maximum(dot(g2, W2) + b2, 0.0)
  z = dot(h1, Wl1_ref[0:D, :])
  z += dot(h2, Wl1_ref[D:2 * D, :])
  z += dot(jnp.abs(h1 - h2), Wl1_ref[2 * D:3 * D, :])
  z = jnp.maximum(z + bl1_ref[...], 0.0)
  hc_ref[...] = dot(z, Wl2_ref[...]) + bl2_ref[...]
  hm_ref[...] = jnp.sum(z * WmT_ref[...], axis=1, keepdims=True) + bm_ref[...]


_head_kernel = pl.pallas_call(
    _head_body,
    out_shape=(jax.ShapeDtypeStruct((N_PAIRS, N_CLASSES), f32),
               jax.ShapeDtypeStruct((N_PAIRS, 1), f32)),
)


def kernel(h, edge_index, x1, x2, W1, b1, W2, b2, Wl1, bl1, Wl2, bl2, Wmse,
           bmse):
  src = edge_index[0]
  dst = edge_index[1]
  npad_e = NW * EPW - N_EDGES
  padidx = (N_NODES +
            (jnp.arange(npad_e, dtype=i32) % (NPAD - N_NODES))).astype(i32)
  srcp = jnp.concatenate([src, padidx]).reshape(NW * NCHUNK, CHUNK)
  dstp = jnp.concatenate([dst, padidx]).reshape(NW * NCHUNK, CHUNK)

  degO, degI = _deg_kernel(srcp, dstp)
  hn1, s_in, s_out = _norm_kernel(h, degO, degI)
  parts1 = _mp_kernel(hn1, srcp, dstp)
  hn2 = _layer1_kernel(parts1, s_in, s_out, W1, b1.reshape(1, D))
  g1p, g2p, sx1, sx2 = _mp2_kernel(hn2, srcp, dstp, x1, x2, s_in)
  h_c, h_mse = _head_kernel(g1p, g2p, sx1, sx2, W2,
                            b2.reshape(1, D), Wl1, bl1.reshape(1, D), Wl2,
                            bl2.reshape(1, N_CLASSES), Wmse.reshape(1, D),
                            bmse.reshape(1, 1))
  return (h_c, h_mse)
